# baseline (speedup 1.0000x reference)
import functools

import jax
import jax.numpy as jnp
from jax.experimental import pallas as pl
from jax.experimental.pallas import tpu as pltpu

B, C, T, K = 8, 768, 2048, 1024
TCOL = 512
NT = T // TCOL


def _prep_kernel(w_ref, cb_ref, wb_ref, cbb_ref, cnorm_ref):
    cb = cb_ref[...]
    wb_ref[...] = w_ref[...].astype(jnp.bfloat16)
    cbb_ref[...] = cb.astype(jnp.bfloat16)
    cnorm_ref[...] = jnp.sum(cb * cb, axis=1, keepdims=True)


def _codes_kernel(wb_ref, pb_ref, cbb_ref, cnorm_ref, ssl_ref, out_ref,
                  buf0, buf1, sem0, sem1):
    b = pl.program_id(0)
    bufs = (buf0, buf1)
    sems = (sem0, sem1)

    def start_copy(bb, chunk, par):
        pltpu.make_async_copy(
            ssl_ref.at[bb, :, pl.ds(chunk * TCOL, TCOL)],
            bufs[par], sems[par]).start()

    @pl.when(b == 0)
    def _prologue():
        start_copy(0, 0, 0)

    for j in range(NT):
        par = j % 2
        pltpu.make_async_copy(
            ssl_ref.at[b, :, pl.ds(j * TCOL, TCOL)],
            bufs[par], sems[par]).wait()
        if j + 1 < NT:
            start_copy(b, j + 1, (j + 1) % 2)
        else:
            @pl.when(b < B - 1)
            def _next_batch():
                start_copy(b + 1, 0, 0)
        s = bufs[par][...].astype(jnp.bfloat16)
        x = jnp.dot(wb_ref[...], s,
                    preferred_element_type=jnp.float32) + pb_ref[...]
        xb = x.astype(jnp.bfloat16)
        znorm = jnp.sum(x * x, axis=0, keepdims=True)
        dots = jnp.dot(cbb_ref[...], xb,
                       preferred_element_type=jnp.float32)
        d = (znorm - 2.0 * dots) + cnorm_ref[...]
        i = jnp.argmin(d, axis=0).astype(jnp.int32)
        out_ref[0, 0, j * TCOL:(j + 1) * TCOL] = i


@functools.partial(jax.jit, static_argnames=())
def kernel(ssl_content, proj_w, proj_b, codebook):
    wb, cbb, cnorm = pl.pallas_call(
        _prep_kernel,
        out_shape=(
            jax.ShapeDtypeStruct((C, C), jnp.bfloat16),
            jax.ShapeDtypeStruct((K, C), jnp.bfloat16),
            jax.ShapeDtypeStruct((K, 1), jnp.float32),
        ),
    )(proj_w, codebook)

    codes = pl.pallas_call(
        _codes_kernel,
        grid=(B,),
        in_specs=[
            pl.BlockSpec((C, C), lambda b: (0, 0)),
            pl.BlockSpec((C, 1), lambda b: (0, 0)),
            pl.BlockSpec((K, C), lambda b: (0, 0)),
            pl.BlockSpec((K, 1), lambda b: (0, 0)),
            pl.BlockSpec(memory_space=pl.ANY),
        ],
        out_specs=pl.BlockSpec((1, 1, T), lambda b: (b, 0, 0)),
        out_shape=jax.ShapeDtypeStruct((B, 1, T), jnp.int32),
        scratch_shapes=[
            pltpu.VMEM((C, TCOL), jnp.float32),
            pltpu.VMEM((C, TCOL), jnp.float32),
            pltpu.SemaphoreType.DMA,
            pltpu.SemaphoreType.DMA,
        ],
        compiler_params=pltpu.CompilerParams(
            dimension_semantics=("arbitrary",)),
    )(wb, proj_b.reshape(C, 1), cbb, cnorm, ssl_content)

    return codes.reshape(B, T)
